# baseline (speedup 1.0000x reference)
import jax
import jax.numpy as jnp
from jax.experimental import pallas as pl
from jax.experimental.pallas import tpu as pltpu

N = 10000
NFEAT = 128
NHID = 128
CLASSES = 64

BI = 200
NI = N // BI
BIB = 1000
NIB = N // BIB


def _layer0_kernel(adj_ref, x_ref, w0_ref, h1_ref, adjq_ref, z_ref):
    i = pl.program_id(0)

    @pl.when(i == 0)
    def _():
        z_ref[...] = jnp.dot(x_ref[...], w0_ref[...],
                             preferred_element_type=jnp.float32)

    a = adj_ref[...]
    adjq_ref[...] = (jnp.round(a * 255.0) - 128.0).astype(jnp.int8)
    row = jnp.dot(a, z_ref[...], preferred_element_type=jnp.float32)
    h1_ref[...] = jnp.maximum(row, 0.0).astype(jnp.bfloat16)


def _layers12_kernel(adjq_ref, h1_ref, w1_ref, w2_ref, out_ref,
                     h2_ref, zb_ref, corr_ref):
    l = pl.program_id(0)
    i = pl.program_id(1)

    @pl.when(i == 0)
    def _():
        @pl.when(l == 0)
        def _():
            z = jnp.dot(h1_ref[...], w1_ref[...].astype(jnp.bfloat16),
                        preferred_element_type=jnp.float32)
            zb_ref[...] = (z * (1.0 / 255.0)).astype(jnp.bfloat16)
            corr_ref[...] = (128.0 / 255.0) * jnp.sum(z, axis=0, keepdims=True)

        @pl.when(l == 1)
        def _():
            z = jnp.dot(h2_ref[...], w2_ref[...].astype(jnp.bfloat16),
                        preferred_element_type=jnp.float32)
            zb_ref[:, :CLASSES] = (z * (1.0 / 255.0)).astype(jnp.bfloat16)
            corr_ref[:, :CLASSES] = (128.0 / 255.0) * jnp.sum(
                z, axis=0, keepdims=True)

    q = adjq_ref[...].astype(jnp.bfloat16)
    row = (jnp.dot(q, zb_ref[...], preferred_element_type=jnp.float32)
           + corr_ref[...])

    @pl.when(l == 0)
    def _():
        h2_ref[pl.ds(i * BIB, BIB), :] = jnp.maximum(row, 0.0).astype(
            jnp.bfloat16)

    @pl.when(l == 1)
    def _():
        out_ref[...] = row[:, :CLASSES]


@jax.jit
def kernel(adj, x, W0, W1, W2):
    h1, adjq = pl.pallas_call(
        _layer0_kernel,
        grid=(NI,),
        in_specs=[
            pl.BlockSpec((BI, N), lambda i: (i, 0)),
            pl.BlockSpec((N, NFEAT), lambda i: (0, 0)),
            pl.BlockSpec((NFEAT, NHID), lambda i: (0, 0)),
        ],
        out_specs=[
            pl.BlockSpec((BI, NHID), lambda i: (i, 0)),
            pl.BlockSpec((BI, N), lambda i: (i, 0)),
        ],
        out_shape=[
            jax.ShapeDtypeStruct((N, NHID), jnp.bfloat16),
            jax.ShapeDtypeStruct((N, N), jnp.int8),
        ],
        scratch_shapes=[
            pltpu.VMEM((N, NHID), jnp.float32),
        ],
        compiler_params=pltpu.CompilerParams(
            dimension_semantics=("arbitrary",),
        ),
    )(adj, x, W0)

    return pl.pallas_call(
        _layers12_kernel,
        grid=(2, NIB),
        in_specs=[
            pl.BlockSpec((BIB, N), lambda l, i: (i, 0)),
            pl.BlockSpec((N, NHID), lambda l, i: (0, 0)),
            pl.BlockSpec((NHID, NHID), lambda l, i: (0, 0)),
            pl.BlockSpec((NHID, CLASSES), lambda l, i: (0, 0)),
        ],
        out_specs=pl.BlockSpec((BIB, CLASSES), lambda l, i: (i, 0)),
        out_shape=jax.ShapeDtypeStruct((N, CLASSES), jnp.float32),
        scratch_shapes=[
            pltpu.VMEM((N, NHID), jnp.bfloat16),
            pltpu.VMEM((N, NHID), jnp.bfloat16),
            pltpu.VMEM((1, NHID), jnp.float32),
        ],
        compiler_params=pltpu.CompilerParams(
            dimension_semantics=("arbitrary", "arbitrary"),
        ),
    )(adjq, h1, W1, W2)


# speedup vs baseline: 1.0000x; 1.0000x over previous
import jax
import jax.numpy as jnp
from jax.experimental import pallas as pl
from jax.experimental.pallas import tpu as pltpu

N = 10000
NFEAT = 128
NHID = 128
CLASSES = 64

BI = 200
NI = N // BI
BIB = 1000
NIB = N // BIB


def _layer0_kernel(adj_ref, x_ref, w0_ref, h1_ref, adjq_ref, z_ref):
    i = pl.program_id(0)

    @pl.when(i == 0)
    def _():
        z_ref[...] = jnp.dot(x_ref[...], w0_ref[...],
                             preferred_element_type=jnp.float32)

    a = adj_ref[...]
    adjq_ref[...] = (jnp.round(a * 255.0) - 128.0).astype(jnp.int8)
    row = jnp.dot(a, z_ref[...], preferred_element_type=jnp.float32)
    h1_ref[...] = jnp.maximum(row, 0.0).astype(jnp.bfloat16)


def _layers12_kernel(adjq_ref, h1_ref, w1_ref, w2_ref, out_ref,
                     h2_ref, zb_ref, corr_ref):
    l = pl.program_id(0)
    i = pl.program_id(1)

    @pl.when(i == 0)
    def _():
        @pl.when(l == 0)
        def _():
            z = jnp.dot(h1_ref[...], w1_ref[...].astype(jnp.bfloat16),
                        preferred_element_type=jnp.float32)
            zb_ref[...] = (z * (1.0 / 255.0)).astype(jnp.bfloat16)
            corr_ref[...] = (128.0 / 255.0) * jnp.sum(z, axis=0, keepdims=True)

        @pl.when(l == 1)
        def _():
            z = jnp.dot(h2_ref[...], w2_ref[...],
                        preferred_element_type=jnp.float32)
            zb_ref[:, :CLASSES] = (z * (1.0 / 255.0)).astype(jnp.bfloat16)
            corr_ref[:, :CLASSES] = (128.0 / 255.0) * jnp.sum(
                z, axis=0, keepdims=True)

    q = adjq_ref[...].astype(jnp.bfloat16)
    row = (jnp.dot(q, zb_ref[...], preferred_element_type=jnp.float32)
           + corr_ref[...])

    @pl.when(l == 0)
    def _():
        h2_ref[pl.ds(i * BIB, BIB), :] = jnp.maximum(row, 0.0)

    @pl.when(l == 1)
    def _():
        out_ref[...] = row[:, :CLASSES]


@jax.jit
def kernel(adj, x, W0, W1, W2):
    h1, adjq = pl.pallas_call(
        _layer0_kernel,
        grid=(NI,),
        in_specs=[
            pl.BlockSpec((BI, N), lambda i: (i, 0)),
            pl.BlockSpec((N, NFEAT), lambda i: (0, 0)),
            pl.BlockSpec((NFEAT, NHID), lambda i: (0, 0)),
        ],
        out_specs=[
            pl.BlockSpec((BI, NHID), lambda i: (i, 0)),
            pl.BlockSpec((BI, N), lambda i: (i, 0)),
        ],
        out_shape=[
            jax.ShapeDtypeStruct((N, NHID), jnp.bfloat16),
            jax.ShapeDtypeStruct((N, N), jnp.int8),
        ],
        scratch_shapes=[
            pltpu.VMEM((N, NHID), jnp.float32),
        ],
        compiler_params=pltpu.CompilerParams(
            dimension_semantics=("arbitrary",),
        ),
    )(adj, x, W0)

    return pl.pallas_call(
        _layers12_kernel,
        grid=(2, NIB),
        in_specs=[
            pl.BlockSpec((BIB, N), lambda l, i: (i, 0)),
            pl.BlockSpec((N, NHID), lambda l, i: (0, 0)),
            pl.BlockSpec((NHID, NHID), lambda l, i: (0, 0)),
            pl.BlockSpec((NHID, CLASSES), lambda l, i: (0, 0)),
        ],
        out_specs=pl.BlockSpec((BIB, CLASSES), lambda l, i: (i, 0)),
        out_shape=jax.ShapeDtypeStruct((N, CLASSES), jnp.float32),
        scratch_shapes=[
            pltpu.VMEM((N, NHID), jnp.float32),
            pltpu.VMEM((N, NHID), jnp.bfloat16),
            pltpu.VMEM((1, NHID), jnp.float32),
        ],
        compiler_params=pltpu.CompilerParams(
            dimension_semantics=("arbitrary", "arbitrary"),
        ),
    )(adjq, h1, W1, W2)


# speedup vs baseline: 1.0190x; 1.0190x over previous
import jax
import jax.numpy as jnp
from jax.experimental import pallas as pl
from jax.experimental.pallas import tpu as pltpu

N = 10000
NFEAT = 128
NHID = 128
CLASSES = 64

BI = 400
NI = N // BI
BIB = 1000
NIB = N // BIB


def _layer0_kernel(adj_ref, x_ref, w0_ref, h1_ref, adjq_ref, z_ref):
    i = pl.program_id(0)

    @pl.when(i == 0)
    def _():
        z_ref[...] = jnp.dot(x_ref[...], w0_ref[...],
                             preferred_element_type=jnp.float32)

    a = adj_ref[...]
    adjq_ref[...] = (jnp.round(a * 255.0) - 128.0).astype(jnp.int8)
    row = jnp.dot(a, z_ref[...], preferred_element_type=jnp.float32)
    h1_ref[...] = jnp.maximum(row, 0.0).astype(jnp.bfloat16)


def _layer1_kernel(adjq_ref, h1_ref, w1_ref, h2_ref, zb_ref, corr_ref):
    i = pl.program_id(0)

    @pl.when(i == 0)
    def _():
        z = jnp.dot(h1_ref[...], w1_ref[...].astype(jnp.bfloat16),
                    preferred_element_type=jnp.float32)
        zb_ref[...] = (z * (1.0 / 255.0)).astype(jnp.bfloat16)
        corr_ref[...] = (128.0 / 255.0) * jnp.sum(z, axis=0, keepdims=True)

    q = adjq_ref[...].astype(jnp.bfloat16)
    row = (jnp.dot(q, zb_ref[...], preferred_element_type=jnp.float32)
           + corr_ref[...])
    h2_ref[...] = jnp.maximum(row, 0.0)


def _layer2_kernel(adjq_ref, h2_ref, w2_ref, out_ref, zb_ref, corr_ref):
    i = pl.program_id(0)

    @pl.when(i == 0)
    def _():
        z = jnp.dot(h2_ref[...], w2_ref[...],
                    preferred_element_type=jnp.float32)
        zb_ref[...] = (z * (1.0 / 255.0)).astype(jnp.bfloat16)
        corr_ref[...] = (128.0 / 255.0) * jnp.sum(z, axis=0, keepdims=True)

    q = adjq_ref[...].astype(jnp.bfloat16)
    row = (jnp.dot(q, zb_ref[...], preferred_element_type=jnp.float32)
           + corr_ref[...])
    out_ref[...] = row


@jax.jit
def kernel(adj, x, W0, W1, W2):
    h1, adjq = pl.pallas_call(
        _layer0_kernel,
        grid=(NI,),
        in_specs=[
            pl.BlockSpec((BI, N), lambda i: (i, 0)),
            pl.BlockSpec((N, NFEAT), lambda i: (0, 0)),
            pl.BlockSpec((NFEAT, NHID), lambda i: (0, 0)),
        ],
        out_specs=[
            pl.BlockSpec((BI, NHID), lambda i: (i, 0)),
            pl.BlockSpec((BI, N), lambda i: (i, 0)),
        ],
        out_shape=[
            jax.ShapeDtypeStruct((N, NHID), jnp.bfloat16),
            jax.ShapeDtypeStruct((N, N), jnp.int8),
        ],
        scratch_shapes=[
            pltpu.VMEM((N, NHID), jnp.float32),
        ],
        compiler_params=pltpu.CompilerParams(
            dimension_semantics=("arbitrary",),
        ),
    )(adj, x, W0)

    h2 = pl.pallas_call(
        _layer1_kernel,
        grid=(NIB,),
        in_specs=[
            pl.BlockSpec((BIB, N), lambda i: (i, 0)),
            pl.BlockSpec((N, NHID), lambda i: (0, 0)),
            pl.BlockSpec((NHID, NHID), lambda i: (0, 0)),
        ],
        out_specs=pl.BlockSpec((BIB, NHID), lambda i: (i, 0)),
        out_shape=jax.ShapeDtypeStruct((N, NHID), jnp.float32),
        scratch_shapes=[
            pltpu.VMEM((N, NHID), jnp.bfloat16),
            pltpu.VMEM((1, NHID), jnp.float32),
        ],
        compiler_params=pltpu.CompilerParams(
            dimension_semantics=("arbitrary",),
        ),
    )(adjq, h1, W1)

    return pl.pallas_call(
        _layer2_kernel,
        grid=(NIB,),
        in_specs=[
            pl.BlockSpec((BIB, N), lambda i: (i, 0)),
            pl.BlockSpec((N, NHID), lambda i: (0, 0)),
            pl.BlockSpec((NHID, CLASSES), lambda i: (0, 0)),
        ],
        out_specs=pl.BlockSpec((BIB, CLASSES), lambda i: (i, 0)),
        out_shape=jax.ShapeDtypeStruct((N, CLASSES), jnp.float32),
        scratch_shapes=[
            pltpu.VMEM((N, CLASSES), jnp.bfloat16),
            pltpu.VMEM((1, CLASSES), jnp.float32),
        ],
        compiler_params=pltpu.CompilerParams(
            dimension_semantics=("arbitrary",),
        ),
    )(adjq, h2, W2)


# speedup vs baseline: 1.0197x; 1.0007x over previous
import jax
import jax.numpy as jnp
from jax.experimental import pallas as pl
from jax.experimental.pallas import tpu as pltpu

N = 10000
NFEAT = 128
NHID = 128
CLASSES = 64

BI = 400
NI = N // BI
BIB = 1000
NIB = N // BIB


def _layer0_kernel(adj_ref, x_ref, w0_ref, h1_ref, adjq_ref, z_ref):
    i = pl.program_id(0)

    @pl.when(i == 0)
    def _():
        z_ref[...] = jnp.dot(x_ref[...], w0_ref[...],
                             preferred_element_type=jnp.float32)

    a = adj_ref[...]
    adjq_ref[...] = (jnp.round(a * 255.0) - 128.0).astype(jnp.int8)
    row = jnp.dot(a, z_ref[...], preferred_element_type=jnp.float32)
    h1_ref[...] = jnp.maximum(row, 0.0).astype(jnp.bfloat16)


def _layer1_kernel(adjq_ref, h1_ref, w1_ref, h2_ref, zb_ref, corr_ref):
    i = pl.program_id(0)

    @pl.when(i == 0)
    def _():
        z = jnp.dot(h1_ref[...], w1_ref[...].astype(jnp.bfloat16),
                    preferred_element_type=jnp.float32)
        zb_ref[...] = (z * (1.0 / 255.0)).astype(jnp.bfloat16)
        corr_ref[...] = (128.0 / 255.0) * jnp.sum(z, axis=0, keepdims=True)

    q = adjq_ref[...].astype(jnp.bfloat16)
    row = (jnp.dot(q, zb_ref[...], preferred_element_type=jnp.float32)
           + corr_ref[...])
    h2_ref[...] = jnp.maximum(row, 0.0)


def _layer2_kernel(adjq_ref, h2_ref, w2_ref, out_ref, zb_ref, corr_ref):
    i = pl.program_id(0)

    @pl.when(i == 0)
    def _():
        z = jnp.dot(h2_ref[...], w2_ref[...],
                    preferred_element_type=jnp.float32)
        zb_ref[:, :CLASSES] = (z * (1.0 / 255.0)).astype(jnp.bfloat16)
        zb_ref[:, CLASSES:] = jnp.zeros((N, NHID - CLASSES), jnp.bfloat16)
        corr_ref[:, :CLASSES] = (128.0 / 255.0) * jnp.sum(
            z, axis=0, keepdims=True)
        corr_ref[:, CLASSES:] = jnp.zeros((1, NHID - CLASSES), jnp.float32)

    q = adjq_ref[...].astype(jnp.bfloat16)
    row = (jnp.dot(q, zb_ref[...], preferred_element_type=jnp.float32)
           + corr_ref[...])
    out_ref[...] = row[:, :CLASSES]


@jax.jit
def kernel(adj, x, W0, W1, W2):
    h1, adjq = pl.pallas_call(
        _layer0_kernel,
        grid=(NI,),
        in_specs=[
            pl.BlockSpec((BI, N), lambda i: (i, 0)),
            pl.BlockSpec((N, NFEAT), lambda i: (0, 0)),
            pl.BlockSpec((NFEAT, NHID), lambda i: (0, 0)),
        ],
        out_specs=[
            pl.BlockSpec((BI, NHID), lambda i: (i, 0)),
            pl.BlockSpec((BI, N), lambda i: (i, 0)),
        ],
        out_shape=[
            jax.ShapeDtypeStruct((N, NHID), jnp.bfloat16),
            jax.ShapeDtypeStruct((N, N), jnp.int8),
        ],
        scratch_shapes=[
            pltpu.VMEM((N, NHID), jnp.float32),
        ],
        compiler_params=pltpu.CompilerParams(
            dimension_semantics=("arbitrary",),
        ),
    )(adj, x, W0)

    h2 = pl.pallas_call(
        _layer1_kernel,
        grid=(NIB,),
        in_specs=[
            pl.BlockSpec((BIB, N), lambda i: (i, 0)),
            pl.BlockSpec((N, NHID), lambda i: (0, 0)),
            pl.BlockSpec((NHID, NHID), lambda i: (0, 0)),
        ],
        out_specs=pl.BlockSpec((BIB, NHID), lambda i: (i, 0)),
        out_shape=jax.ShapeDtypeStruct((N, NHID), jnp.float32),
        scratch_shapes=[
            pltpu.VMEM((N, NHID), jnp.bfloat16),
            pltpu.VMEM((1, NHID), jnp.float32),
        ],
        compiler_params=pltpu.CompilerParams(
            dimension_semantics=("arbitrary",),
        ),
    )(adjq, h1, W1)

    return pl.pallas_call(
        _layer2_kernel,
        grid=(NIB,),
        in_specs=[
            pl.BlockSpec((BIB, N), lambda i: (i, 0)),
            pl.BlockSpec((N, NHID), lambda i: (0, 0)),
            pl.BlockSpec((NHID, CLASSES), lambda i: (0, 0)),
        ],
        out_specs=pl.BlockSpec((BIB, CLASSES), lambda i: (i, 0)),
        out_shape=jax.ShapeDtypeStruct((N, CLASSES), jnp.float32),
        scratch_shapes=[
            pltpu.VMEM((N, NHID), jnp.bfloat16),
            pltpu.VMEM((1, NHID), jnp.float32),
        ],
        compiler_params=pltpu.CompilerParams(
            dimension_semantics=("arbitrary",),
        ),
    )(adjq, h2, W2)


# speedup vs baseline: 1.0409x; 1.0207x over previous
import jax
import jax.numpy as jnp
from jax.experimental import pallas as pl
from jax.experimental.pallas import tpu as pltpu

N = 10000
NFEAT = 128
NHID = 128
CLASSES = 64

BI = 400
NI = N // BI
BIB = 1000
NIB = N // BIB


def _layer0_kernel(adj_ref, x_ref, w0_ref, h1_ref, adjq_ref, z_ref):
    i = pl.program_id(0)

    @pl.when(i == 0)
    def _():
        z_ref[...] = jnp.dot(x_ref[...], w0_ref[...],
                             preferred_element_type=jnp.float32)

    a = adj_ref[...]
    adjq_ref[...] = (jnp.round(a * 255.0) - 128.0).astype(jnp.int8)
    row = jnp.dot(a, z_ref[...], preferred_element_type=jnp.float32)
    h1_ref[...] = jnp.maximum(row, 0.0).astype(jnp.bfloat16)


def _layers12_kernel(adjq_ref, h1_ref, w1_ref, w2_ref, out_ref,
                     h2_ref, zb_ref, corr_ref):
    l = pl.program_id(0)
    i = pl.program_id(1)

    @pl.when(i == 0)
    def _():
        @pl.when(l == 0)
        def _():
            z = jnp.dot(h1_ref[...], w1_ref[...].astype(jnp.bfloat16),
                        preferred_element_type=jnp.float32)
            zb_ref[...] = (z * (1.0 / 255.0)).astype(jnp.bfloat16)
            corr_ref[...] = (128.0 / 255.0) * jnp.sum(z, axis=0, keepdims=True)

        @pl.when(l == 1)
        def _():
            z = jnp.dot(h2_ref[...], w2_ref[...],
                        preferred_element_type=jnp.float32)
            zb_ref[:, :CLASSES] = (z * (1.0 / 255.0)).astype(jnp.bfloat16)
            corr_ref[:, :CLASSES] = (128.0 / 255.0) * jnp.sum(
                z, axis=0, keepdims=True)

    q = adjq_ref[...].astype(jnp.bfloat16)
    row = (jnp.dot(q, zb_ref[...], preferred_element_type=jnp.float32)
           + corr_ref[...])

    @pl.when(l == 0)
    def _():
        h2_ref[pl.ds(i * BIB, BIB), :] = jnp.maximum(row, 0.0)

    @pl.when(l == 1)
    def _():
        out_ref[...] = row[:, :CLASSES]


@jax.jit
def kernel(adj, x, W0, W1, W2):
    h1, adjq = pl.pallas_call(
        _layer0_kernel,
        grid=(NI,),
        in_specs=[
            pl.BlockSpec((BI, N), lambda i: (i, 0)),
            pl.BlockSpec((N, NFEAT), lambda i: (0, 0)),
            pl.BlockSpec((NFEAT, NHID), lambda i: (0, 0)),
        ],
        out_specs=[
            pl.BlockSpec((BI, NHID), lambda i: (i, 0)),
            pl.BlockSpec((BI, N), lambda i: (i, 0)),
        ],
        out_shape=[
            jax.ShapeDtypeStruct((N, NHID), jnp.bfloat16),
            jax.ShapeDtypeStruct((N, N), jnp.int8),
        ],
        scratch_shapes=[
            pltpu.VMEM((N, NHID), jnp.float32),
        ],
        compiler_params=pltpu.CompilerParams(
            dimension_semantics=("arbitrary",),
        ),
    )(adj, x, W0)

    return pl.pallas_call(
        _layers12_kernel,
        grid=(2, NIB),
        in_specs=[
            pl.BlockSpec((BIB, N), lambda l, i: (i, 0)),
            pl.BlockSpec((N, NHID), lambda l, i: (0, 0)),
            pl.BlockSpec((NHID, NHID), lambda l, i: (0, 0)),
            pl.BlockSpec((NHID, CLASSES), lambda l, i: (0, 0)),
        ],
        out_specs=pl.BlockSpec((BIB, CLASSES), lambda l, i: (i, 0)),
        out_shape=jax.ShapeDtypeStruct((N, CLASSES), jnp.float32),
        scratch_shapes=[
            pltpu.VMEM((N, NHID), jnp.float32),
            pltpu.VMEM((N, NHID), jnp.bfloat16),
            pltpu.VMEM((1, NHID), jnp.float32),
        ],
        compiler_params=pltpu.CompilerParams(
            dimension_semantics=("arbitrary", "arbitrary"),
        ),
    )(adjq, h1, W1, W2)
